# initial kernel scaffold (unmeasured)
import jax
import jax.numpy as jnp
from jax import lax
from jax.experimental import pallas as pl
from jax.experimental.pallas import tpu as pltpu

N_DEV = 32
LOG2_DEV = 5
EPS = 1e-5


def kernel(x, gamma, beta):
    m, n = x.shape
    n_global = n * N_DEV

    def body(x_ref, g_ref, b_ref, o_ref, acc_ref, recv_ref, send_sems, recv_sems):
        my = lax.axis_index("i")

        xv = x_ref[:, :]
        acc_ref[:, 0:1] = jnp.sum(xv, axis=1, keepdims=True)
        acc_ref[:, 1:2] = jnp.sum(xv * xv, axis=1, keepdims=True)

        for s in range(LOG2_DEV):
            partner = my ^ (1 << s)
            rdma = pltpu.make_async_remote_copy(
                src_ref=acc_ref,
                dst_ref=recv_ref.at[s],
                send_sem=send_sems.at[s],
                recv_sem=recv_sems.at[s],
                device_id=(partner,),
                device_id_type=pl.DeviceIdType.MESH,
            )
            rdma.start()
            rdma.wait()
            acc_ref[:, :] = acc_ref[:, :] + recv_ref[s]

        mean = acc_ref[:, 0:1] * (1.0 / n_global)
        var = acc_ref[:, 1:2] * (1.0 / n_global) - mean * mean
        inv = lax.rsqrt(var + EPS)
        o_ref[:, :] = g_ref[0:1, :] * ((xv - mean) * inv) + b_ref[0:1, :]

    return pl.pallas_call(
        body,
        out_shape=jax.ShapeDtypeStruct((m, n), x.dtype),
        in_specs=[
            pl.BlockSpec(memory_space=pltpu.VMEM),
            pl.BlockSpec(memory_space=pltpu.VMEM),
            pl.BlockSpec(memory_space=pltpu.VMEM),
        ],
        out_specs=pl.BlockSpec(memory_space=pltpu.VMEM),
        scratch_shapes=[
            pltpu.VMEM((m, 2), jnp.float32),
            pltpu.VMEM((LOG2_DEV, m, 2), jnp.float32),
            pltpu.SemaphoreType.DMA((LOG2_DEV,)),
            pltpu.SemaphoreType.DMA((LOG2_DEV,)),
        ],
        compiler_params=pltpu.CompilerParams(collective_id=0),
    )(x, gamma.reshape(1, n), beta.reshape(1, n))


# baseline (device time: 57289 ns/iter reference)
import jax
import jax.numpy as jnp
from jax import lax
from jax.experimental import pallas as pl
from jax.experimental.pallas import tpu as pltpu

N_DEV = 32
LOG2_DEV = 5
EPS = 1e-5


def kernel(x, gamma, beta):
    m, n = x.shape
    n_global = n * N_DEV

    def body(x_ref, g_ref, b_ref, o_ref, acc_ref, recv_ref, send_sems, recv_sems):
        my = lax.axis_index("i")

        xv = x_ref[:, :]
        acc_ref[:, 0:1] = jnp.sum(xv, axis=1, keepdims=True)
        acc_ref[:, 1:2] = jnp.sum(xv * xv, axis=1, keepdims=True)

        for s in range(LOG2_DEV):
            partner = my ^ (1 << s)
            rdma = pltpu.make_async_remote_copy(
                src_ref=acc_ref,
                dst_ref=recv_ref.at[s],
                send_sem=send_sems.at[s],
                recv_sem=recv_sems.at[s],
                device_id=(partner,),
                device_id_type=pl.DeviceIdType.MESH,
            )
            rdma.start()
            rdma.wait()
            acc_ref[:, :] = acc_ref[:, :] + recv_ref[s]

        mean = acc_ref[:, 0:1] * (1.0 / n_global)
        var = acc_ref[:, 1:2] * (1.0 / n_global) - mean * mean
        inv = lax.rsqrt(var + EPS)
        o_ref[:, :] = g_ref[0:1, :] * ((xv - mean) * inv) + b_ref[0:1, :]

    return pl.pallas_call(
        body,
        out_shape=jax.ShapeDtypeStruct((m, n), x.dtype),
        in_specs=[
            pl.BlockSpec(memory_space=pltpu.VMEM),
            pl.BlockSpec(memory_space=pltpu.VMEM),
            pl.BlockSpec(memory_space=pltpu.VMEM),
        ],
        out_specs=pl.BlockSpec(memory_space=pltpu.VMEM),
        scratch_shapes=[
            pltpu.VMEM((m, 2), jnp.float32),
            pltpu.VMEM((LOG2_DEV, m, 2), jnp.float32),
            pltpu.SemaphoreType.DMA((LOG2_DEV,)),
            pltpu.SemaphoreType.DMA((LOG2_DEV,)),
        ],
    )(x, gamma.reshape(1, n), beta.reshape(1, n))


# device time: 28897 ns/iter; 1.9825x vs baseline; 1.9825x over previous
import os

import jax
import jax.numpy as jnp
from jax import lax
from jax.experimental import pallas as pl
from jax.experimental.pallas import tpu as pltpu

N_DEV = 32
N_STAGES = int(os.environ.get("KERNEL_STAGES", "5"))
EPS = 1e-5


def kernel(x, gamma, beta):
    m, n = x.shape
    n_global = n * N_DEV

    def body(x_ref, g_ref, b_ref, o_ref, acc_ref, recv_ref, send_sems, recv_sems):
        my = lax.axis_index("i")

        xv = x_ref[:, :]
        acc_ref[0, :] = jnp.sum(xv, axis=1)
        acc_ref[1, :] = jnp.sum(xv * xv, axis=1)

        for s in range(N_STAGES):
            partner = my ^ (1 << s)
            rdma = pltpu.make_async_remote_copy(
                src_ref=acc_ref,
                dst_ref=recv_ref.at[s],
                send_sem=send_sems.at[s],
                recv_sem=recv_sems.at[s],
                device_id=(partner,),
                device_id_type=pl.DeviceIdType.MESH,
            )
            rdma.start()
            rdma.wait()
            acc_ref[:, :] = acc_ref[:, :] + recv_ref[s]

        stats = jnp.transpose(acc_ref[:, :])
        mean = stats[:, 0:1] * (1.0 / n_global)
        var = stats[:, 1:2] * (1.0 / n_global) - mean * mean
        inv = lax.rsqrt(var + EPS)
        o_ref[:, :] = g_ref[0:1, :] * ((xv - mean) * inv) + b_ref[0:1, :]

    return pl.pallas_call(
        body,
        out_shape=jax.ShapeDtypeStruct((m, n), x.dtype),
        in_specs=[
            pl.BlockSpec(memory_space=pltpu.VMEM),
            pl.BlockSpec(memory_space=pltpu.VMEM),
            pl.BlockSpec(memory_space=pltpu.VMEM),
        ],
        out_specs=pl.BlockSpec(memory_space=pltpu.VMEM),
        scratch_shapes=[
            pltpu.VMEM((2, m), jnp.float32),
            pltpu.VMEM((max(N_STAGES, 1), 2, m), jnp.float32),
            pltpu.SemaphoreType.DMA((max(N_STAGES, 1),)),
            pltpu.SemaphoreType.DMA((max(N_STAGES, 1),)),
        ],
    )(x, gamma.reshape(1, n), beta.reshape(1, n))


# device time: 17178 ns/iter; 3.3350x vs baseline; 1.6822x over previous
import jax
import jax.numpy as jnp
from jax import lax
from jax.experimental import pallas as pl
from jax.experimental.pallas import tpu as pltpu

N_DEV = 32
EPS = 1e-5


def kernel(x, gamma, beta):
    m, n = x.shape
    n_global = n * N_DEV

    def body(x_ref, g_ref, b_ref, o_ref, comm_ref, send_sems, recv_sems):
        my = lax.axis_index("i")

        barrier_sem = pltpu.get_barrier_semaphore()
        for d in range(1, N_DEV):
            pl.semaphore_signal(
                barrier_sem, inc=1,
                device_id=(my ^ d,), device_id_type=pl.DeviceIdType.MESH,
            )

        xv = x_ref[:, :]
        comm_ref[0, 0, :] = jnp.sum(xv, axis=1)
        comm_ref[0, 1, :] = jnp.sum(xv * xv, axis=1)

        pl.semaphore_wait(barrier_sem, N_DEV - 1)

        rdmas = []
        for d in range(1, N_DEV):
            rdma = pltpu.make_async_remote_copy(
                src_ref=comm_ref.at[0],
                dst_ref=comm_ref.at[d],
                send_sem=send_sems.at[d],
                recv_sem=recv_sems.at[d],
                device_id=(my ^ d,),
                device_id_type=pl.DeviceIdType.MESH,
            )
            rdma.start()
            rdmas.append(rdma)
        for rdma in rdmas:
            rdma.wait()

        total = jnp.sum(comm_ref[:, :, :], axis=0)
        stats = jnp.transpose(total)
        mean = stats[:, 0:1] * (1.0 / n_global)
        var = stats[:, 1:2] * (1.0 / n_global) - mean * mean
        inv = lax.rsqrt(var + EPS)
        o_ref[:, :] = g_ref[0:1, :] * ((xv - mean) * inv) + b_ref[0:1, :]

    return pl.pallas_call(
        body,
        out_shape=jax.ShapeDtypeStruct((m, n), x.dtype),
        in_specs=[
            pl.BlockSpec(memory_space=pltpu.VMEM),
            pl.BlockSpec(memory_space=pltpu.VMEM),
            pl.BlockSpec(memory_space=pltpu.VMEM),
        ],
        out_specs=pl.BlockSpec(memory_space=pltpu.VMEM),
        scratch_shapes=[
            pltpu.VMEM((N_DEV, 2, m), jnp.float32),
            pltpu.SemaphoreType.DMA((N_DEV,)),
            pltpu.SemaphoreType.DMA((N_DEV,)),
        ],
        compiler_params=pltpu.CompilerParams(collective_id=0),
    )(x, gamma.reshape(1, n), beta.reshape(1, n))
